# baseline (speedup 1.0000x reference)
import jax
import jax.numpy as jnp
from jax import lax
from jax.experimental import pallas as pl
from jax.experimental.pallas import tpu as pltpu
from jax.experimental.pallas import tpu_sc as plsc

_VOCAB = 1000000
_D = 64
_B = 4096
_N = 200

_NC = 2
_NS = 16
_NW = _NC * _NS

_FLAT = _B * _N
_PER_W = _FLAT // _NW
_CHUNK = 256
_SUB = 128
_NSUB = _CHUNK // _SUB
_NCHUNK = _PER_W // _CHUNK
_POSEXT = 768


def _body(x_hbm, table_hbm, posext_hbm, out_hbm, idx_v, gidx_v, rows_v, out_v,
          posext_v, sem):
    wid = lax.axis_index("s") * _NC + lax.axis_index("c")

    pltpu.sync_copy(posext_hbm, posext_v)

    def group_body(g, o):
        iv16 = idx_v[g >> 3, pl.ds((g & 7) * 16, 16)]
        p16 = posext_v[pl.ds(o + g * 16, 16)]
        for j in range(16):
            s_idx = iv16[j]
            h = (s_idx & 1) * _D
            bm = jnp.full((16,), jnp.where(s_idx == 0, 0.0, 1.0), jnp.float32)
            bp = jnp.full((16,), p16[j], jnp.float32)
            r = g * 16 + j
            for c in range(_D // 16):
                v = rows_v[r, pl.ds(h + c * 16, 16)]
                out_v[r, pl.ds(c * 16, 16)] = v * bm + bp
        return o

    def chunk_body(ci, _):
        cb = wid * _PER_W + ci * _CHUNK
        pltpu.sync_copy(x_hbm.at[pl.ds(wid * (_PER_W // _SUB) + ci * _NSUB, _NSUB)],
                        idx_v)
        for gr in range(_NSUB):
            for gc in range(_SUB // 16):
                sl = pl.ds(gc * 16, 16)
                gidx_v[gr, sl] = idx_v[gr, sl] >> 1
        descs = [
            pltpu.async_copy(
                table_hbm.at[gidx_v.at[j]], rows_v.at[pl.ds(j * _SUB, _SUB)], sem
            )
            for j in range(_NSUB)
        ]
        for d in descs:
            d.wait()
        lax.fori_loop(0, _CHUNK // 16, group_body, cb % _N)
        pltpu.sync_copy(out_v, out_hbm.at[pl.ds(cb, _CHUNK)])
        return 0

    lax.fori_loop(0, _NCHUNK, chunk_body, 0)


@jax.jit
def kernel(X, table, pos):
    x_flat = X.reshape(_FLAT // _SUB, _SUB)
    table128 = table.reshape(_VOCAB // 2, 2 * _D)
    p = pos[:, 0]
    posext = jnp.concatenate([p, p, p, p[: _POSEXT - 3 * _N]])

    mesh = plsc.VectorSubcoreMesh(core_axis_name="c", subcore_axis_name="s")
    out = pl.kernel(
        _body,
        out_type=jax.ShapeDtypeStruct((_FLAT, _D), jnp.float32),
        mesh=mesh,
        compiler_params=pltpu.CompilerParams(needs_layout_passes=False),
        scratch_types=[
            pltpu.VMEM((_NSUB, _SUB), jnp.int32),
            pltpu.VMEM((_NSUB, _SUB), jnp.int32),
            pltpu.VMEM((_CHUNK, 2 * _D), jnp.float32),
            pltpu.VMEM((_CHUNK, _D), jnp.float32),
            pltpu.VMEM((_POSEXT,), jnp.float32),
            pltpu.SemaphoreType.DMA,
        ],
    )(x_flat, table128, posext)
    return out.reshape(_B, _N, _D)


# speedup vs baseline: 1.1352x; 1.1352x over previous
import jax
import jax.numpy as jnp
from jax import lax
from jax.experimental import pallas as pl
from jax.experimental.pallas import tpu as pltpu
from jax.experimental.pallas import tpu_sc as plsc

_VOCAB = 1000000
_D = 64
_B = 4096
_N = 200

_NC = 2
_NS = 16
_NW = _NC * _NS

_ROWS_W = _B // _NW
_NG = _N // 16
_TAIL = _NG * 16 - 8


def _body(x_hbm, table_hbm, pos_hbm, out_hbm, idx_v, gidx_v, rows_v, out_v,
          pos_v, gsem, ssem):
    wid = lax.axis_index("s") * _NC + lax.axis_index("c")

    pltpu.sync_copy(pos_hbm, pos_v)

    def stage(c, buf):
        b = wid * _ROWS_W + c
        pltpu.sync_copy(x_hbm.at[b], idx_v.at[buf])
        for o in list(range(0, _NG * 16, 16)) + [_TAIL]:
            sl = pl.ds(o, 16)
            gidx_v[buf, sl] = idx_v[buf, sl] >> 1
        pltpu.async_copy(table_hbm.at[gidx_v.at[buf].at[pl.ds(0, 128)]],
                         rows_v.at[buf].at[pl.ds(0, 128)], gsem.at[buf])
        pltpu.async_copy(table_hbm.at[gidx_v.at[buf].at[pl.ds(128, _N - 128)]],
                         rows_v.at[buf].at[pl.ds(128, _N - 128)], gsem.at[buf])

    def wait_gather(buf):
        pltpu.make_async_copy(table_hbm.at[gidx_v.at[buf].at[pl.ds(0, 128)]],
                              rows_v.at[buf].at[pl.ds(0, 128)], gsem.at[buf]).wait()
        pltpu.make_async_copy(table_hbm.at[gidx_v.at[buf].at[pl.ds(128, _N - 128)]],
                              rows_v.at[buf].at[pl.ds(128, _N - 128)],
                              gsem.at[buf]).wait()

    def compute(c, buf):
        wait_gather(buf)

        def do_rows(o, j0):
            iv16 = idx_v[buf, pl.ds(o, 16)]
            p16 = pos_v[pl.ds(o, 16)]
            for j in range(j0, 16):
                s = iv16[j]
                h = (s & 1) * _D
                bm = jnp.full((16,), jnp.where(s == 0, 0.0, 1.0), jnp.float32)
                bp = jnp.full((16,), p16[j], jnp.float32)
                for cc in range(_D // 16):
                    v = rows_v[buf, o + j, pl.ds(h + cc * 16, 16)]
                    out_v[buf, o + j, pl.ds(cc * 16, 16)] = v * bm + bp

        def group_body(g, _):
            do_rows(g * 16, 0)
            return 0

        lax.fori_loop(0, _NG, group_body, 0)
        do_rows(_TAIL, 8)

    def scatter(c, buf):
        b = wid * _ROWS_W + c
        pltpu.async_copy(out_v.at[buf], out_hbm.at[b], ssem.at[buf])

    def wait_scatter(c, buf):
        b = wid * _ROWS_W + c
        pltpu.make_async_copy(out_v.at[buf], out_hbm.at[b], ssem.at[buf]).wait()

    stage(0, 0)
    stage(1, 1)

    def pair_body(k, _):
        c0 = k * 2

        @pl.when(k > 0)
        def _():
            wait_scatter(c0 - 2, 0)
        compute(c0, 0)
        scatter(c0, 0)

        @pl.when(k < _ROWS_W // 2 - 1)
        def _():
            stage(c0 + 2, 0)

        @pl.when(k > 0)
        def _():
            wait_scatter(c0 - 1, 1)
        compute(c0 + 1, 1)
        scatter(c0 + 1, 1)

        @pl.when(k < _ROWS_W // 2 - 1)
        def _():
            stage(c0 + 3, 1)
        return 0

    lax.fori_loop(0, _ROWS_W // 2, pair_body, 0)
    wait_scatter(_ROWS_W - 2, 0)
    wait_scatter(_ROWS_W - 1, 1)


@jax.jit
def kernel(X, table, pos):
    table128 = table.reshape(_VOCAB // 2, 2 * _D)
    mesh = plsc.VectorSubcoreMesh(core_axis_name="c", subcore_axis_name="s")
    out = pl.kernel(
        _body,
        out_type=jax.ShapeDtypeStruct((_B, _N, _D), jnp.float32),
        mesh=mesh,
        compiler_params=pltpu.CompilerParams(needs_layout_passes=False),
        scratch_types=[
            pltpu.VMEM((2, _N), jnp.int32),
            pltpu.VMEM((2, _N), jnp.int32),
            pltpu.VMEM((2, _N, 2 * _D), jnp.float32),
            pltpu.VMEM((2, _N, _D), jnp.float32),
            pltpu.VMEM((_N,), jnp.float32),
            pltpu.SemaphoreType.DMA((2,)),
            pltpu.SemaphoreType.DMA((2,)),
        ],
    )(X, table128, pos[:, 0])
    return out


# speedup vs baseline: 1.6761x; 1.4765x over previous
import jax
import jax.numpy as jnp
from jax import lax
from jax.experimental import pallas as pl
from jax.experimental.pallas import tpu as pltpu
from jax.experimental.pallas import tpu_sc as plsc

_VOCAB = 1000000
_D = 64
_B = 4096
_N = 200

_NC = 2
_NS = 16
_NW = _NC * _NS

_ROWS_W = _B // _NW
_NG = _N // 16
_TAIL = _NG * 16 - 8


def _body(x_hbm, table_hbm, pos_hbm, out_hbm, idx_v, rows_v, out_v, pos_v,
          gsem, ssem):
    wid = lax.axis_index("s") * _NC + lax.axis_index("c")

    pltpu.sync_copy(pos_hbm, pos_v)

    def stage(c, buf):
        b = wid * _ROWS_W + c
        pltpu.sync_copy(x_hbm.at[b], idx_v.at[buf])
        pltpu.async_copy(table_hbm.at[idx_v.at[buf].at[pl.ds(0, 128)]],
                         rows_v.at[buf].at[pl.ds(0, 128)], gsem.at[buf])
        pltpu.async_copy(table_hbm.at[idx_v.at[buf].at[pl.ds(128, _N - 128)]],
                         rows_v.at[buf].at[pl.ds(128, _N - 128)], gsem.at[buf])

    def wait_gather(buf):
        pltpu.make_async_copy(table_hbm.at[idx_v.at[buf].at[pl.ds(0, 128)]],
                              rows_v.at[buf].at[pl.ds(0, 128)], gsem.at[buf]).wait()
        pltpu.make_async_copy(table_hbm.at[idx_v.at[buf].at[pl.ds(128, _N - 128)]],
                              rows_v.at[buf].at[pl.ds(128, _N - 128)],
                              gsem.at[buf]).wait()

    def compute(c, buf):
        wait_gather(buf)

        def do_rows(o, j0):
            iv16 = idx_v[buf, pl.ds(o, 16)]
            p16 = pos_v[pl.ds(o, 16)]
            for j in range(j0, 16):
                s = iv16[j]
                bm = jnp.full((16,), jnp.where(s == 0, 0.0, 1.0), jnp.float32)
                bp = jnp.full((16,), p16[j], jnp.float32)
                for cc in range(_D // 16):
                    v = rows_v[buf, o + j, pl.ds(cc * 16, 16)]
                    out_v[buf, o + j, pl.ds(cc * 16, 16)] = v * bm + bp

        def group_body(g, _):
            do_rows(g * 16, 0)
            return 0

        lax.fori_loop(0, _NG, group_body, 0, unroll=2)
        do_rows(_TAIL, 8)

    def scatter(c, buf):
        b = wid * _ROWS_W + c
        pltpu.async_copy(out_v.at[buf], out_hbm.at[pl.ds(b * _N, _N)], ssem.at[buf])

    def wait_scatter(c, buf):
        b = wid * _ROWS_W + c
        pltpu.make_async_copy(out_v.at[buf], out_hbm.at[pl.ds(b * _N, _N)],
                              ssem.at[buf]).wait()

    stage(0, 0)
    stage(1, 1)

    def pair_body(k, _):
        c0 = k * 2

        @pl.when(k > 0)
        def _():
            wait_scatter(c0 - 2, 0)
        compute(c0, 0)
        scatter(c0, 0)

        @pl.when(k < _ROWS_W // 2 - 1)
        def _():
            stage(c0 + 2, 0)

        @pl.when(k > 0)
        def _():
            wait_scatter(c0 - 1, 1)
        compute(c0 + 1, 1)
        scatter(c0 + 1, 1)

        @pl.when(k < _ROWS_W // 2 - 1)
        def _():
            stage(c0 + 3, 1)
        return 0

    lax.fori_loop(0, _ROWS_W // 2, pair_body, 0)
    wait_scatter(_ROWS_W - 2, 0)
    wait_scatter(_ROWS_W - 1, 1)


@jax.jit
def kernel(X, table, pos):
    table_pad = jnp.pad(table, ((0, 0), (0, _D)))
    mesh = plsc.VectorSubcoreMesh(core_axis_name="c", subcore_axis_name="s")
    out = pl.kernel(
        _body,
        out_type=jax.ShapeDtypeStruct((_B * _N, _D), jnp.float32),
        mesh=mesh,
        compiler_params=pltpu.CompilerParams(needs_layout_passes=False),
        scratch_types=[
            pltpu.VMEM((2, _N), jnp.int32),
            pltpu.VMEM((2, _N, 2 * _D), jnp.float32),
            pltpu.VMEM((2, _N, _D), jnp.float32),
            pltpu.VMEM((_N,), jnp.float32),
            pltpu.SemaphoreType.DMA((2,)),
            pltpu.SemaphoreType.DMA((2,)),
        ],
    )(X, table_pad, pos[:, 0])
    return out.reshape(_B, _N, _D)


# speedup vs baseline: 1.6930x; 1.0101x over previous
import jax
import jax.numpy as jnp
from jax import lax
from jax.experimental import pallas as pl
from jax.experimental.pallas import tpu as pltpu
from jax.experimental.pallas import tpu_sc as plsc

_VOCAB = 1000000
_D = 64
_B = 4096
_N = 200

_NC = 2
_NS = 16
_NW = _NC * _NS

_ROWS_W = _B // _NW
_NG = _N // 16
_TAIL = _NG * 16 - 8

_GATHER_DNUMS = lax.GatherDimensionNumbers(
    offset_dims=(), collapsed_slice_dims=(0,), start_index_map=(0,)
)


def _lane_broadcast(v16, j):
    idx = jnp.full((16, 1), j, jnp.int32)
    return lax.gather(
        v16, idx, _GATHER_DNUMS, (1,),
        mode=lax.GatherScatterMode.PROMISE_IN_BOUNDS,
    )


def _body(x_hbm, table_hbm, pos_hbm, out_hbm, idx_v, rows_v, out_v, pos_v,
          gsem, ssem):
    wid = lax.axis_index("s") * _NC + lax.axis_index("c")

    pltpu.sync_copy(pos_hbm, pos_v)

    def stage(c, buf):
        b = wid * _ROWS_W + c
        pltpu.sync_copy(x_hbm.at[b], idx_v.at[buf])
        pltpu.async_copy(table_hbm.at[idx_v.at[buf].at[pl.ds(0, 128)]],
                         rows_v.at[buf].at[pl.ds(0, 128)], gsem.at[buf])
        pltpu.async_copy(table_hbm.at[idx_v.at[buf].at[pl.ds(128, _N - 128)]],
                         rows_v.at[buf].at[pl.ds(128, _N - 128)], gsem.at[buf])

    def wait_gather(buf):
        pltpu.make_async_copy(table_hbm.at[idx_v.at[buf].at[pl.ds(0, 128)]],
                              rows_v.at[buf].at[pl.ds(0, 128)], gsem.at[buf]).wait()
        pltpu.make_async_copy(table_hbm.at[idx_v.at[buf].at[pl.ds(128, _N - 128)]],
                              rows_v.at[buf].at[pl.ds(128, _N - 128)],
                              gsem.at[buf]).wait()

    def compute(c, buf):
        wait_gather(buf)

        def do_rows(o, j0):
            iv16 = idx_v[buf, pl.ds(o, 16)]
            p16 = pos_v[pl.ds(o, 16)]
            m16 = jnp.where(iv16 == 0, jnp.float32(0.0), jnp.float32(1.0))
            for j in range(j0, 16):
                bm = _lane_broadcast(m16, j)
                bp = _lane_broadcast(p16, j)
                for cc in range(_D // 16):
                    v = rows_v[buf, o + j, pl.ds(cc * 16, 16)]
                    out_v[buf, o + j, pl.ds(cc * 16, 16)] = v * bm + bp

        def group_body(g, _):
            do_rows(g * 16, 0)
            return 0

        lax.fori_loop(0, _NG, group_body, 0, unroll=2)
        do_rows(_TAIL, 8)

    def scatter(c, buf):
        b = wid * _ROWS_W + c
        pltpu.async_copy(out_v.at[buf], out_hbm.at[pl.ds(b * _N, _N)], ssem.at[buf])

    def wait_scatter(c, buf):
        b = wid * _ROWS_W + c
        pltpu.make_async_copy(out_v.at[buf], out_hbm.at[pl.ds(b * _N, _N)],
                              ssem.at[buf]).wait()

    stage(0, 0)
    stage(1, 1)

    def pair_body(k, _):
        c0 = k * 2

        @pl.when(k > 0)
        def _():
            wait_scatter(c0 - 2, 0)
        compute(c0, 0)
        scatter(c0, 0)

        @pl.when(k < _ROWS_W // 2 - 1)
        def _():
            stage(c0 + 2, 0)

        @pl.when(k > 0)
        def _():
            wait_scatter(c0 - 1, 1)
        compute(c0 + 1, 1)
        scatter(c0 + 1, 1)

        @pl.when(k < _ROWS_W // 2 - 1)
        def _():
            stage(c0 + 3, 1)
        return 0

    lax.fori_loop(0, _ROWS_W // 2, pair_body, 0)
    wait_scatter(_ROWS_W - 2, 0)
    wait_scatter(_ROWS_W - 1, 1)


@jax.jit
def kernel(X, table, pos):
    table_pad = jnp.pad(table, ((0, 0), (0, _D)))
    mesh = plsc.VectorSubcoreMesh(core_axis_name="c", subcore_axis_name="s")
    out = pl.kernel(
        _body,
        out_type=jax.ShapeDtypeStruct((_B * _N, _D), jnp.float32),
        mesh=mesh,
        compiler_params=pltpu.CompilerParams(needs_layout_passes=False),
        scratch_types=[
            pltpu.VMEM((2, _N), jnp.int32),
            pltpu.VMEM((2, _N, 2 * _D), jnp.float32),
            pltpu.VMEM((2, _N, _D), jnp.float32),
            pltpu.VMEM((_N,), jnp.float32),
            pltpu.SemaphoreType.DMA((2,)),
            pltpu.SemaphoreType.DMA((2,)),
        ],
    )(X, table_pad, pos[:, 0])
    return out.reshape(_B, _N, _D)
